# baseline (speedup 1.0000x reference)
import functools

import jax
import jax.numpy as jnp
from jax import lax
from jax.experimental import pallas as pl
from jax.experimental.pallas import tpu as pltpu
from jax.experimental.pallas import tpu_sc as plsc

_B, _C, _N, _K = 4, 64, 16384, 16
_OUT = 64
_BN = _B * _N
_KH = _K // 2
_KO2 = _KH * _OUT
_W2 = _OUT // 2
_NW = 32
_PTS_W = _BN // _NW
_CH = 64
_HR = _CH * _KH
_ROWS = 2 * _HR
_NCHUNK = _PTS_W // _CH
_GSUB = 128
_NG = _HR // _GSUB
_BM = 1024


def _mm_body(t_ref, w_ref, b_ref, y_ref):
    res = (
        jnp.dot(t_ref[...], w_ref[...], preferred_element_type=jnp.float32)
        + b_ref[...]
    )
    u = jax.lax.bitcast_convert_type(res, jnp.uint32) + jnp.uint32(0x8000)
    lo = u[:, : _KO2 // 2] >> 16
    hi = u[:, _KO2 // 2:] & jnp.uint32(0xFFFF0000)
    y_ref[...] = lo | hi


def _matmul_y(table, wall, ball):
    return pl.pallas_call(
        _mm_body,
        grid=(_BN // _BM,),
        in_specs=[
            pl.BlockSpec((_BM, _C), lambda m: (m, 0)),
            pl.BlockSpec((_C, _KO2), lambda m: (0, 0)),
            pl.BlockSpec((1, _KO2), lambda m: (0, 0)),
        ],
        out_specs=pl.BlockSpec((_BM, _KO2 // 2), lambda m: (m, 0)),
        out_shape=jax.ShapeDtypeStruct((_BN, _KO2 // 2), jnp.uint32),
    )(table, wall, ball)


def _sc_gather_reduce(yra, yrb, idx2a, idx2b):
    mesh = plsc.VectorSubcoreMesh(core_axis_name="c", subcore_axis_name="s")

    @functools.partial(
        pl.kernel,
        out_type=jax.ShapeDtypeStruct((_BN, _W2), jnp.uint32),
        mesh=mesh,
        scratch_types=[
            pltpu.VMEM((2, _HR), jnp.int32),
            pltpu.VMEM((2, _HR), jnp.int32),
            pltpu.VMEM((2, _ROWS, _W2), jnp.uint32),
            pltpu.VMEM((_CH, _W2), jnp.uint32),
            pltpu.SemaphoreType.DMA,
            pltpu.SemaphoreType.DMA,
        ],
        compiler_params=pltpu.CompilerParams(
            use_tc_tiling_on_sc=False, needs_layout_passes=False
        ),
    )
    def k(ya_hbm, yb_hbm, ia_hbm, ib_hbm, out_hbm, iva, ivb, rows_v, out_v,
          sem0, sem1):
        wid = lax.axis_index("s") * 2 + lax.axis_index("c")
        base0 = wid * _PTS_W
        sems = (sem0, sem1)

        def fire(buf, base_pt, sem):
            pltpu.sync_copy(ia_hbm.at[pl.ds(base_pt * _KH, _HR)], iva.at[buf])
            pltpu.sync_copy(ib_hbm.at[pl.ds(base_pt * _KH, _HR)], ivb.at[buf])
            for g in range(_NG):
                pltpu.async_copy(
                    ya_hbm.at[iva.at[buf, pl.ds(g * _GSUB, _GSUB)]],
                    rows_v.at[buf, pl.ds(g * _GSUB, _GSUB)],
                    sem,
                )
            for g in range(_NG):
                pltpu.async_copy(
                    yb_hbm.at[ivb.at[buf, pl.ds(g * _GSUB, _GSUB)]],
                    rows_v.at[buf, pl.ds(_HR + g * _GSUB, _GSUB)],
                    sem,
                )

        def compute(buf, base_pt):
            pltpu.make_async_copy(
                ya_hbm.at[iva.at[buf]], rows_v.at[buf], sems[buf]
            ).wait()

            zero = jnp.zeros((32,), jnp.bfloat16)

            def pt_body(p, c2):
                row0 = p * _KH
                for c0 in range(0, _W2, 16):
                    s = [
                        plsc.bitcast(
                            rows_v[buf, row0 + kk, pl.ds(c0, 16)],
                            jnp.bfloat16,
                        )
                        for kk in range(_KH)
                    ] + [
                        plsc.bitcast(
                            rows_v[buf, _HR + row0 + kk, pl.ds(c0, 16)],
                            jnp.bfloat16,
                        )
                        for kk in range(_KH)
                    ]
                    while len(s) > 1:
                        s = [a + b for a, b in zip(s[::2], s[1::2])]
                    out_v[p, pl.ds(c0, 16)] = plsc.bitcast(
                        jnp.maximum(s[0], zero), jnp.uint32
                    )
                return c2

            lax.fori_loop(0, _CH, pt_body, 0)
            pltpu.sync_copy(out_v, out_hbm.at[pl.ds(base_pt, _CH)])

        fire(0, base0, sem0)

        def chunk2(ci2, carry):
            base_pt = base0 + ci2 * 2 * _CH
            fire(1, base_pt + _CH, sem1)
            compute(0, base_pt)

            @pl.when(ci2 + 1 < _NCHUNK // 2)
            def _():
                fire(0, base_pt + 2 * _CH, sem0)

            compute(1, base_pt + _CH)
            return carry

        lax.fori_loop(0, _NCHUNK // 2, chunk2, 0)

    return k(yra, yrb, idx2a, idx2b)


def kernel(feature, idx, W, b):
    table = feature.transpose(0, 2, 1).reshape(_BN, _C)
    w4 = (
        W.reshape(_OUT, _K, _C)
        .transpose(2, 1, 0)
        .reshape(_C, _K, 2, _OUT // 2)
        .transpose(0, 2, 1, 3)
    )
    wall_a = w4[:, :, :_KH, :].reshape(_C, _KO2)
    wall_b = w4[:, :, _KH:, :].reshape(_C, _KO2)
    ball = jnp.broadcast_to(
        (b / _K).reshape(2, 1, _OUT // 2), (2, _KH, _OUT // 2)
    ).reshape(1, _KO2)

    ya = _matmul_y(table, wall_a, ball)
    yb = _matmul_y(table, wall_b, ball)
    yra = ya.reshape(_BN * _KH, _W2)
    yrb = yb.reshape(_BN * _KH, _W2)

    idx32 = idx[:, :, :_K].astype(jnp.int32)
    boff = (jnp.arange(_B, dtype=jnp.int32) * _N)[:, None, None]
    koff = jnp.arange(_KH, dtype=jnp.int32)[None, None, :]
    idx2a = ((idx32[:, :, :_KH] + boff) * _KH + koff).reshape(_BN * _KH)
    idx2b = ((idx32[:, :, _KH:] + boff) * _KH + koff).reshape(_BN * _KH)

    out2 = _sc_gather_reduce(yra, yrb, idx2a, idx2b)
    out_bf = jax.lax.bitcast_convert_type(out2, jnp.bfloat16)
    out64 = jnp.concatenate([out_bf[:, :, 0], out_bf[:, :, 1]], axis=-1)
    return (
        out64.reshape(_B, _N, _OUT).transpose(0, 2, 1).astype(jnp.float32)
    )


# speedup vs baseline: 1.1971x; 1.1971x over previous
import functools

import jax
import jax.numpy as jnp
from jax import lax
from jax.experimental import pallas as pl
from jax.experimental.pallas import tpu as pltpu
from jax.experimental.pallas import tpu_sc as plsc

_B, _C, _N, _K = 4, 64, 16384, 16
_OUT = 64
_BN = _B * _N
_KO = _K * _OUT
_NW = 32
_PTS_W = _BN // _NW
_CH = 64
_ROWS = _CH * _K
_NCHUNK = _PTS_W // _CH
_GSUB = 128
_NG = _ROWS // _GSUB
_BM = 1024


def _mm_body(t_ref, w_ref, b_ref, y_ref):
    res = (
        jnp.dot(t_ref[...], w_ref[...], preferred_element_type=jnp.float32)
        + b_ref[...]
    )
    u = jax.lax.bitcast_convert_type(res, jnp.uint32) + jnp.uint32(0x8000)
    lo = u[:, : _KO // 2] >> 16
    hi = u[:, _KO // 2:] & jnp.uint32(0xFFFF0000)
    y_ref[...] = lo | hi


def _matmul_y(table, wall, ball):
    return pl.pallas_call(
        _mm_body,
        grid=(_BN // _BM,),
        in_specs=[
            pl.BlockSpec((_BM, _C), lambda m: (m, 0)),
            pl.BlockSpec((_C, _KO), lambda m: (0, 0)),
            pl.BlockSpec((1, _KO), lambda m: (0, 0)),
        ],
        out_specs=pl.BlockSpec((_BM, _KO // 2), lambda m: (m, 0)),
        out_shape=jax.ShapeDtypeStruct((_BN, _KO // 2), jnp.uint32),
    )(table, wall, ball)


_W2 = _OUT // 2


def _sc_gather_reduce(yr, idx2):
    mesh = plsc.VectorSubcoreMesh(core_axis_name="c", subcore_axis_name="s")

    @functools.partial(
        pl.kernel,
        out_type=jax.ShapeDtypeStruct((_BN, _W2), jnp.uint32),
        mesh=mesh,
        scratch_types=[
            pltpu.VMEM((2, _ROWS), jnp.int32),
            pltpu.VMEM((2, _ROWS, _W2), jnp.uint32),
            pltpu.VMEM((_CH, _W2), jnp.uint32),
            pltpu.SemaphoreType.DMA,
            pltpu.SemaphoreType.DMA,
        ],
        compiler_params=pltpu.CompilerParams(
            use_tc_tiling_on_sc=False, needs_layout_passes=False
        ),
    )
    def k(yr_hbm, idx_hbm, out_hbm, idx_v, rows_v, out_v, sem0, sem1):
        wid = lax.axis_index("s") * 2 + lax.axis_index("c")
        base0 = wid * _PTS_W
        sems = (sem0, sem1)

        def fire(buf, base_pt, sem):
            pltpu.sync_copy(
                idx_hbm.at[pl.ds(base_pt * _K, _ROWS)], idx_v.at[buf]
            )
            for g in range(_NG):
                pltpu.async_copy(
                    yr_hbm.at[idx_v.at[buf, pl.ds(g * _GSUB, _GSUB)]],
                    rows_v.at[buf, pl.ds(g * _GSUB, _GSUB)],
                    sem,
                )

        def compute(buf, base_pt):
            pltpu.make_async_copy(
                yr_hbm.at[idx_v.at[buf]], rows_v.at[buf], sems[buf]
            ).wait()

            zero = jnp.zeros((32,), jnp.bfloat16)

            def pt_body(p, c2):
                row0 = p * _K
                for c0 in range(0, _W2, 16):
                    s = [
                        plsc.bitcast(
                            rows_v[buf, row0 + kk, pl.ds(c0, 16)],
                            jnp.bfloat16,
                        )
                        for kk in range(_K)
                    ]
                    while len(s) > 1:
                        s = [a + b for a, b in zip(s[::2], s[1::2])]
                    out_v[p, pl.ds(c0, 16)] = plsc.bitcast(
                        jnp.maximum(s[0], zero), jnp.uint32
                    )
                return c2

            lax.fori_loop(0, _CH, pt_body, 0)
            pltpu.sync_copy(out_v, out_hbm.at[pl.ds(base_pt, _CH)])

        fire(0, base0, sem0)

        def chunk2(ci2, carry):
            base_pt = base0 + ci2 * 2 * _CH
            fire(1, base_pt + _CH, sem1)
            compute(0, base_pt)

            @pl.when(ci2 + 1 < _NCHUNK // 2)
            def _():
                fire(0, base_pt + 2 * _CH, sem0)

            compute(1, base_pt + _CH)
            return carry

        lax.fori_loop(0, _NCHUNK // 2, chunk2, 0)

    return k(yr, idx2)


def kernel(feature, idx, W, b):
    table = feature.transpose(0, 2, 1).reshape(_BN, _C)
    wall = (
        W.reshape(_OUT, _K, _C)
        .transpose(2, 1, 0)
        .reshape(_C, _K, 2, _OUT // 2)
        .transpose(0, 2, 1, 3)
        .reshape(_C, _KO)
    )
    ball = jnp.broadcast_to(
        (b / _K).reshape(2, 1, _OUT // 2), (2, _K, _OUT // 2)
    ).reshape(1, _KO)

    y = _matmul_y(table, wall, ball)
    yr = y.reshape(_BN * _K, _W2)

    idx32 = idx[:, :, :_K].astype(jnp.int32)
    boff = (jnp.arange(_B, dtype=jnp.int32) * _N)[:, None, None]
    koff = jnp.arange(_K, dtype=jnp.int32)[None, None, :]
    idx2 = ((idx32 + boff) * _K + koff).reshape(_BN * _K)

    out2 = _sc_gather_reduce(yr, idx2)
    out_bf = jax.lax.bitcast_convert_type(out2, jnp.bfloat16)
    out64 = jnp.concatenate([out_bf[:, :, 0], out_bf[:, :, 1]], axis=-1)
    return (
        out64.reshape(_B, _N, _OUT).transpose(0, 2, 1).astype(jnp.float32)
    )
